# baseline (speedup 1.0000x reference)
import math

import jax
import jax.numpy as jnp
from jax.experimental import pallas as pl
from jax.experimental.pallas import tpu as pltpu

_K = 2
_CF = 1.25
_MIN_CAP = 4
_E = 64
_T = 32768
_B = 2048
_NB = _T // _B
_CH = 128
_NCH = _B // _CH


def _capacity(num_tokens, num_experts):
    cap = math.floor(_K * _CF * num_tokens / num_experts)
    cap += cap % 2
    return max(cap, _MIN_CAP)

_CAP = float(_capacity(_T, _E))


def _pass1_body(lt_ref, kept0_ref, v2_ref, r1_ref, i2_ref, c0_ref,
                aux_ref, carry, me_acc):
    i = pl.program_id(0)

    @pl.when(i == 0)
    def _init():
        carry[...] = jnp.zeros_like(carry)
        me_acc[...] = jnp.zeros_like(me_acc)

    r = jax.lax.broadcasted_iota(jnp.int32, (_CH, _CH), 0)
    c = jax.lax.broadcasted_iota(jnp.int32, (_CH, _CH), 1)
    triu = (r <= c).astype(jnp.bfloat16)
    revi = 63 - jax.lax.broadcasted_iota(jnp.int32, (_E, _CH), 0)
    ones_e = jnp.ones((1, _E), jnp.float32)

    run0 = carry[:_E, :]
    run1 = carry[_E:, :]
    acc = me_acc[...]
    for j in range(_NCH):
        lt = lt_ref[:, pl.ds(j * _CH, _CH)]
        ex = jnp.exp(lt)
        s = jnp.dot(ones_e, ex, preferred_element_type=jnp.float32)
        rs = 1.0 / s

        exi = jax.lax.bitcast_convert_type(ex, jnp.int32)
        key = (exi & jnp.int32(-64)) | revi
        pm = jax.lax.bitcast_convert_type(key, jnp.float32)
        v1k = jnp.max(pm, axis=0, keepdims=True)
        oh0 = (pm == v1k)
        pm2 = jnp.where(oh0, 0.0, pm)
        v2k = jnp.max(pm2, axis=0, keepdims=True)
        oh1 = (pm2 == v2k)
        v1ki = jax.lax.bitcast_convert_type(v1k, jnp.int32)
        v2ki = jax.lax.bitcast_convert_type(v2k, jnp.int32)
        i2 = 63 - (v2ki & 63)
        val1 = jax.lax.bitcast_convert_type(
            v1ki & jnp.int32(-64), jnp.float32) * rs
        val2 = jax.lax.bitcast_convert_type(
            v2ki & jnp.int32(-64), jnp.float32) * rs

        cj0 = jnp.dot(oh0.astype(jnp.bfloat16), triu,
                      preferred_element_type=jnp.float32)
        cj1 = jnp.dot(oh1.astype(jnp.bfloat16), triu,
                      preferred_element_type=jnp.float32)
        prod0 = jnp.where(oh0, cj0 + run0, 0.0)
        prod1 = jnp.where(oh1, cj1 + run1, 0.0)
        pos0 = jnp.dot(ones_e, prod0,
                       preferred_element_type=jnp.float32) - 1.0
        r1 = jnp.dot(ones_e, prod1,
                     preferred_element_type=jnp.float32) - 1.0
        run0 = run0 + cj0[:, _CH - 1:_CH]
        run1 = run1 + cj1[:, _CH - 1:_CH]

        keep0 = (pos0 < _CAP).astype(jnp.float32)
        sl = pl.ds(j * _CH, _CH)
        kept0_ref[:, :, sl] = (val1 * keep0).reshape(1, 1, _CH)
        v2_ref[:, :, sl] = val2.reshape(1, 1, _CH)
        r1_ref[:, :, sl] = r1.reshape(1, 1, _CH)
        i2_ref[:, :, sl] = i2.astype(jnp.float32).reshape(1, 1, _CH)

        acc = acc + jnp.dot(ex, rs.T, preferred_element_type=jnp.float32)

    carry[:_E, :] = run0
    carry[_E:, :] = run1
    me_acc[...] = acc

    @pl.when(i == _NB - 1)
    def _tail():
        c0_ref[...] = run0
        t = jnp.float32(_T)
        aux_ref[...] = (jnp.float32(_E) * jnp.sum(
            (acc / t) * (run0 / t))).reshape(1, 1)


def _pass2_body(kept0_ref, v2_ref, r1_ref, i2_ref, c0_ref, out0_ref, out1_ref):
    kept0 = kept0_ref[...]
    v2 = v2_ref[...]
    r1 = r1_ref[...]
    i2 = i2_ref[...]
    c0sel = jnp.zeros_like(r1)
    for e in range(_E):
        c0sel = jnp.where(i2 == float(e), c0_ref[e, 0], c0sel)
    keep1 = ((c0sel + r1) < _CAP).astype(jnp.float32)
    kv1 = v2 * keep1
    denom = kept0 + kv1 + 1e-9
    out0_ref[...] = kept0 / denom
    out1_ref[...] = kv1 / denom


@jax.jit
def kernel(logits):
    lt = logits.T
    tok_spec = pl.BlockSpec((1, 1, _B), lambda i: (i, 0, 0))
    tok_shape = jax.ShapeDtypeStruct((_NB, 1, _B), jnp.float32)
    kept0, v2, r1, i2, c0, aux = pl.pallas_call(
        _pass1_body,
        grid=(_NB,),
        in_specs=[pl.BlockSpec((_E, _B), lambda i: (0, i))],
        out_specs=[tok_spec, tok_spec, tok_spec, tok_spec,
                   pl.BlockSpec((_E, 1), lambda i: (0, 0)),
                   pl.BlockSpec((1, 1), lambda i: (0, 0))],
        out_shape=[
            tok_shape, tok_shape, tok_shape, tok_shape,
            jax.ShapeDtypeStruct((_E, 1), jnp.float32),
            jax.ShapeDtypeStruct((1, 1), jnp.float32),
        ],
        scratch_shapes=[pltpu.VMEM((2 * _E, 1), jnp.float32),
                        pltpu.VMEM((_E, 1), jnp.float32)],
    )(lt)

    shp = (_T // 128, 128)
    out0, out1 = pl.pallas_call(
        _pass2_body,
        out_shape=[jax.ShapeDtypeStruct(shp, jnp.float32)] * 2,
    )(kept0.reshape(shp), v2.reshape(shp), r1.reshape(shp),
      i2.reshape(shp), c0)

    combine = jnp.stack([out0.reshape(-1), out1.reshape(-1)], axis=1)
    return combine, aux[0, 0]
